# baseline (speedup 1.0000x reference)
import functools

import jax
import jax.numpy as jnp
from jax import lax
from jax.experimental import pallas as pl
from jax.experimental.pallas import tpu as pltpu
from jax.experimental.pallas import tpu_sc as plsc

TOTAL = 32768
D = 512
NSEG = 16
L = 16
NCORE = 2
NSUB = 16
COLS = 128
NV = COLS // L
NRG = 8
RROWS = TOTAL // NRG
CH = 128
NCHUNK = RROWS // CH
NBUF = 6

_NEG = -3e38
_LN2 = 0.6931471805599453


def _log16(s):
    xb = lax.bitcast_convert_type(s, jnp.int32)
    e = ((xb >> 23) & 0xFF) - 127
    mb = (xb & 0x7FFFFF) | 0x3F800000
    m = lax.bitcast_convert_type(mb, jnp.float32)
    big = m > jnp.float32(1.4142135)
    m = jnp.where(big, m * jnp.float32(0.5), m)
    e = e + jnp.where(big, jnp.int32(1), jnp.int32(0))
    t = (m - jnp.float32(1.0)) / (m + jnp.float32(1.0))
    t2 = t * t
    p = jnp.float32(1.0) + t2 * (
        jnp.float32(1.0 / 3.0)
        + t2 * (jnp.float32(1.0 / 5.0) + t2 * jnp.float32(1.0 / 7.0))
    )
    return e.astype(jnp.float32) * jnp.float32(_LN2) + jnp.float32(2.0) * t * p


def _seg_bounds(ps_ref, b, base):
    v = ps_ref[pl.ds(b, L)]
    lo = jnp.clip(v[0] - base, 0, CH)
    hi = jnp.clip(v[1] - base, 0, CH)
    return lo, hi


def _row_max(buf, lo, hi):
    neg = tuple(jnp.full((L,), _NEG, jnp.float32) for _ in range(NV))

    def body(i, c):
        return tuple(
            jnp.maximum(c[j], buf[i, pl.ds(j * L, L)]) for j in range(NV)
        )

    return plsc.parallel_loop(lo, hi, 1, unroll=2, carry=neg)(body)


def _row_fused(buf, lo, hi, msh):
    init = tuple(jnp.full((L,), _NEG, jnp.float32) for _ in range(NV)) + tuple(
        jnp.zeros((L,), jnp.float32) for _ in range(NV))

    def body(i, c):
        xs = [buf[i, pl.ds(j * L, L)] for j in range(NV)]
        ms = tuple(jnp.maximum(c[j], xs[j]) for j in range(NV))
        ss = tuple(c[NV + j] + jnp.exp(xs[j] - msh[j]) for j in range(NV))
        return ms + ss

    res = plsc.parallel_loop(lo, hi, 1, unroll=4, carry=init)(body)
    return res[:NV], res[NV:]


def _row_sumexp(buf, lo, hi, ms):
    zero = tuple(jnp.zeros((L,), jnp.float32) for _ in range(NV))

    def body(i, c):
        return tuple(
            c[j] + jnp.exp(buf[i, pl.ds(j * L, L)] - ms[j]) for j in range(NV)
        )

    return plsc.parallel_loop(lo, hi, 1, unroll=2, carry=zero)(body)


def _body(logits_hbm, ps_hbm, out_hbm, bufs, ps_ref, macc, sacc, gm, gs, lse,
          tmp_m, tmp_s, shm, shs, in_sems, out_sems):
    core = lax.axis_index("c")
    sid = lax.axis_index("s")
    cgl = sid // NRG
    rg = lax.rem(sid, jnp.int32(NRG))
    col0 = (core * 2 + cgl) * COLS
    row0 = rg * RROWS

    pltpu.sync_copy(ps_hbm, ps_ref.at[pl.ds(0, NSEG + 1)])

    def in_copy(c, slot):
        return pltpu.make_async_copy(
            logits_hbm.at[pl.ds(row0 + c * CH, CH), pl.ds(col0, COLS)],
            bufs.at[slot],
            in_sems.at[slot],
        )

    def out_copy(c, slot):
        return pltpu.make_async_copy(
            bufs.at[slot],
            out_hbm.at[pl.ds(row0 + c * CH, CH), pl.ds(col0, COLS)],
            out_sems.at[slot - 3],
        )

    def init_b(b, _):
        for j in range(NV):
            macc[b, pl.ds(j * L, L)] = jnp.full((L,), _NEG, jnp.float32)
            sacc[b, pl.ds(j * L, L)] = jnp.zeros((L,), jnp.float32)
        return 0

    lax.fori_loop(0, NSEG, init_b, 0)

    for k in range(NBUF):
        in_copy(jnp.int32(k), jnp.int32(k)).start()

    def p1_chunk(c, _):
        slot = lax.rem(c, jnp.int32(NBUF))
        in_copy(c, slot).wait()
        base = row0 + c * CH
        buf = bufs.at[slot]

        def p1_seg(b, _):
            lo, hi = _seg_bounds(ps_ref, b, base)

            @pl.when(lo < hi)
            def _():
                msh = tuple(
                    jnp.where(macc[b, pl.ds(j * L, L)] < jnp.float32(-1e30),
                              buf[lo, pl.ds(j * L, L)],
                              macc[b, pl.ds(j * L, L)])
                    for j in range(NV)
                )
                m_loc, s_fast = _row_fused(buf, lo, hi, msh)
                overv = m_loc[0] - msh[0]
                for j in range(1, NV):
                    overv = jnp.maximum(overv, m_loc[j] - msh[j])
                nbad = plsc.all_reduce_population_count(
                    overv > jnp.float32(60.0))
                bad = nbad[0] > 0

                @pl.when(jnp.logical_not(bad))
                def _():
                    for j in range(NV):
                        ds = pl.ds(j * L, L)
                        m_old = macc[b, ds]
                        m_new = jnp.maximum(m_old, m_loc[j])
                        sacc[b, ds] = (sacc[b, ds] * jnp.exp(m_old - m_new)
                                       + s_fast[j] * jnp.exp(msh[j] - m_new))
                        macc[b, ds] = m_new

                @pl.when(bad)
                def _():
                    s_loc = _row_sumexp(buf, lo, hi, m_loc)
                    for j in range(NV):
                        ds = pl.ds(j * L, L)
                        m_old = macc[b, ds]
                        m_new = jnp.maximum(m_old, m_loc[j])
                        sacc[b, ds] = (sacc[b, ds] * jnp.exp(m_old - m_new)
                                       + s_loc[j] * jnp.exp(m_loc[j] - m_new))
                        macc[b, ds] = m_new

            return 0

        lax.fori_loop(0, NSEG, p1_seg, 0)

        nxt = c + NBUF

        @pl.when(nxt < NCHUNK)
        def _():
            in_copy(nxt, slot).start()

        return 0

    with jax.named_scope("phase1"):
        lax.fori_loop(0, NCHUNK, p1_chunk, 0)

    for k in range(3):
        in_copy(jnp.int32(k), jnp.int32(k)).start()

    sc_merge = jax.named_scope("scmerge")
    sc_merge.__enter__()
    pltpu.sync_copy(macc, shm.at[cgl, rg])
    pltpu.sync_copy(sacc, shs.at[cgl, rg])
    plsc.subcore_barrier()

    def init_g(b, _):
        for j in range(NV):
            gm[b, pl.ds(j * L, L)] = jnp.full((L,), _NEG, jnp.float32)
            gs[b, pl.ds(j * L, L)] = jnp.zeros((L,), jnp.float32)
        return 0

    lax.fori_loop(0, NSEG, init_g, 0)

    for w in range(NRG):
        pltpu.sync_copy(shm.at[cgl, w], tmp_m)
        pltpu.sync_copy(shs.at[cgl, w], tmp_s)

        def merge_b(b, _):
            for j in range(NV):
                ds = pl.ds(j * L, L)
                m_old = gm[b, ds]
                m_w = tmp_m[b, ds]
                m_new = jnp.maximum(m_old, m_w)
                gs[b, ds] = (gs[b, ds] * jnp.exp(m_old - m_new)
                             + tmp_s[b, ds] * jnp.exp(m_w - m_new))
                gm[b, ds] = m_new
            return 0

        lax.fori_loop(0, NSEG, merge_b, 0)

    def mk_lse(b, _):
        for j in range(NV):
            ds = pl.ds(j * L, L)
            lse[b, ds] = gm[b, ds] + _log16(gs[b, ds])
        return 0

    lax.fori_loop(0, NSEG, mk_lse, 0)
    sc_merge.__exit__(None, None, None)

    def p2_chunk(c, _):
        slot = lax.rem(c, jnp.int32(3))
        oslot = slot + 3
        in_copy(c, slot).wait()

        @pl.when(c >= 3)
        def _():
            out_copy(c - 3, oslot).wait()

        base = row0 + c * CH
        src = bufs.at[slot]
        dst = bufs.at[oslot]

        def p2_seg(b, _):
            lo, hi = _seg_bounds(ps_ref, b, base)

            @pl.when(lo < hi)
            def _():
                lv = tuple(lse[b, pl.ds(j * L, L)] for j in range(NV))

                def apply(i):
                    for j in range(NV):
                        ds = pl.ds(j * L, L)
                        dst[i, ds] = src[i, ds] - lv[j]

                plsc.parallel_loop(lo, hi, 1, unroll=2)(apply)

            return 0

        lax.fori_loop(0, NSEG, p2_seg, 0)
        out_copy(c, oslot).start()

        nxt = c + 3

        @pl.when(nxt < NCHUNK)
        def _():
            in_copy(nxt, slot).start()

        return 0

    with jax.named_scope("phase2"):
        lax.fori_loop(0, NCHUNK, p2_chunk, 0)

    for k in range(3):
        c = NCHUNK - 3 + k
        out_copy(jnp.int32(c), jnp.int32(c % 3 + 3)).wait()


_sc_call = functools.partial(
    pl.kernel,
    out_type=jax.ShapeDtypeStruct((TOTAL, D), jnp.float32),
    mesh=plsc.VectorSubcoreMesh(core_axis_name="c", subcore_axis_name="s"),
    scratch_types=[
        pltpu.VMEM((NBUF, CH, COLS), jnp.float32),
        pltpu.VMEM((32,), jnp.int32),
        pltpu.VMEM((NSEG, COLS), jnp.float32),
        pltpu.VMEM((NSEG, COLS), jnp.float32),
        pltpu.VMEM((NSEG, COLS), jnp.float32),
        pltpu.VMEM((NSEG, COLS), jnp.float32),
        pltpu.VMEM((NSEG, COLS), jnp.float32),
        pltpu.VMEM((NSEG, COLS), jnp.float32),
        pltpu.VMEM((NSEG, COLS), jnp.float32),
        pltpu.VMEM_SHARED((2, NRG, NSEG, COLS), jnp.float32),
        pltpu.VMEM_SHARED((2, NRG, NSEG, COLS), jnp.float32),
        pltpu.SemaphoreType.DMA((NBUF,)),
        pltpu.SemaphoreType.DMA((3,)),
    ],
    compiler_params=pltpu.CompilerParams(needs_layout_passes=False),
)(_body)


@jax.jit
def kernel(logits, prefix_sum):
    return _sc_call(logits, prefix_sum)


# speedup vs baseline: 1.1334x; 1.1334x over previous
import functools

import jax
import jax.numpy as jnp
from jax import lax
from jax.experimental import pallas as pl
from jax.experimental.pallas import tpu as pltpu
from jax.experimental.pallas import tpu_sc as plsc

TOTAL = 32768
D = 512
NSEG = 16
L = 16
NCORE = 2
NSUB = 16
COLS = 128
NV = COLS // L
NRG = 8
RROWS = TOTAL // NRG
CH = 128
NCHUNK = RROWS // CH
NBUF = 6

_NEG = -3e38
_LN2 = 0.6931471805599453


def _log16(s):
    xb = lax.bitcast_convert_type(s, jnp.int32)
    e = ((xb >> 23) & 0xFF) - 127
    mb = (xb & 0x7FFFFF) | 0x3F800000
    m = lax.bitcast_convert_type(mb, jnp.float32)
    big = m > jnp.float32(1.4142135)
    m = jnp.where(big, m * jnp.float32(0.5), m)
    e = e + jnp.where(big, jnp.int32(1), jnp.int32(0))
    t = (m - jnp.float32(1.0)) / (m + jnp.float32(1.0))
    t2 = t * t
    p = jnp.float32(1.0) + t2 * (
        jnp.float32(1.0 / 3.0)
        + t2 * (jnp.float32(1.0 / 5.0) + t2 * jnp.float32(1.0 / 7.0))
    )
    return e.astype(jnp.float32) * jnp.float32(_LN2) + jnp.float32(2.0) * t * p


def _seg_bounds(ps_ref, b, base):
    v = ps_ref[pl.ds(b, L)]
    lo = jnp.clip(v[0] - base, 0, CH)
    hi = jnp.clip(v[1] - base, 0, CH)
    return lo, hi


def _row_max(buf, lo, hi):
    neg = tuple(jnp.full((L,), _NEG, jnp.float32) for _ in range(NV))

    def body(i, c):
        return tuple(
            jnp.maximum(c[j], buf[i, pl.ds(j * L, L)]) for j in range(NV)
        )

    return plsc.parallel_loop(lo, hi, 1, unroll=2, carry=neg)(body)


def _row_fused(buf, lo, hi, msh):
    H = NV // 2
    ms_all = []
    ss_all = []
    for h in range(2):
        j0 = h * H
        init = tuple(jnp.full((L,), _NEG, jnp.float32) for _ in range(H)) + \
            tuple(jnp.zeros((L,), jnp.float32) for _ in range(H))

        def body(i, c, j0=j0):
            xs = [buf[i, pl.ds((j0 + j) * L, L)] for j in range(H)]
            ms = tuple(jnp.maximum(c[j], xs[j]) for j in range(H))
            ss = tuple(c[H + j] + jnp.exp(xs[j] - msh[j0 + j])
                       for j in range(H))
            return ms + ss

        res = plsc.parallel_loop(lo, hi, 1, unroll=2, carry=init)(body)
        ms_all.extend(res[:H])
        ss_all.extend(res[H:])
    return tuple(ms_all), tuple(ss_all)


def _row_sumexp(buf, lo, hi, ms):
    zero = tuple(jnp.zeros((L,), jnp.float32) for _ in range(NV))

    def body(i, c):
        return tuple(
            c[j] + jnp.exp(buf[i, pl.ds(j * L, L)] - ms[j]) for j in range(NV)
        )

    return plsc.parallel_loop(lo, hi, 1, unroll=2, carry=zero)(body)


def _body(logits_hbm, ps_hbm, out_hbm, bufs, ps_ref, macc, sacc, gm, gs, lse,
          tmp_m, tmp_s, shm, shs, in_sems, out_sems):
    core = lax.axis_index("c")
    sid = lax.axis_index("s")
    cgl = sid // NRG
    rg = lax.rem(sid, jnp.int32(NRG))
    col0 = (core * 2 + cgl) * COLS
    row0 = rg * RROWS

    pltpu.sync_copy(ps_hbm, ps_ref.at[pl.ds(0, NSEG + 1)])

    def in_copy(c, slot):
        return pltpu.make_async_copy(
            logits_hbm.at[pl.ds(row0 + c * CH, CH), pl.ds(col0, COLS)],
            bufs.at[slot],
            in_sems.at[slot],
        )

    def out_copy(c, slot):
        return pltpu.make_async_copy(
            bufs.at[slot],
            out_hbm.at[pl.ds(row0 + c * CH, CH), pl.ds(col0, COLS)],
            out_sems.at[slot - 3],
        )

    def init_b(b, _):
        for j in range(NV):
            macc[b, pl.ds(j * L, L)] = jnp.full((L,), _NEG, jnp.float32)
            sacc[b, pl.ds(j * L, L)] = jnp.zeros((L,), jnp.float32)
        return 0

    lax.fori_loop(0, NSEG, init_b, 0)

    for k in range(NBUF):
        in_copy(jnp.int32(k), jnp.int32(k)).start()

    def p1_chunk(c, _):
        slot = lax.rem(c, jnp.int32(NBUF))
        in_copy(c, slot).wait()
        base = row0 + c * CH
        buf = bufs.at[slot]

        def p1_seg(b, _):
            lo, hi = _seg_bounds(ps_ref, b, base)

            @pl.when(lo < hi)
            def _():
                msh = tuple(
                    jnp.where(macc[b, pl.ds(j * L, L)] < jnp.float32(-1e30),
                              buf[lo, pl.ds(j * L, L)],
                              macc[b, pl.ds(j * L, L)])
                    for j in range(NV)
                )
                m_loc, s_fast = _row_fused(buf, lo, hi, msh)
                overv = m_loc[0] - msh[0]
                for j in range(1, NV):
                    overv = jnp.maximum(overv, m_loc[j] - msh[j])
                nbad = plsc.all_reduce_population_count(
                    overv > jnp.float32(60.0))
                bad = nbad[0] > 0

                @pl.when(jnp.logical_not(bad))
                def _():
                    for j in range(NV):
                        ds = pl.ds(j * L, L)
                        m_old = macc[b, ds]
                        m_new = jnp.maximum(m_old, m_loc[j])
                        sacc[b, ds] = (sacc[b, ds] * jnp.exp(m_old - m_new)
                                       + s_fast[j] * jnp.exp(msh[j] - m_new))
                        macc[b, ds] = m_new

                @pl.when(bad)
                def _():
                    s_loc = _row_sumexp(buf, lo, hi, m_loc)
                    for j in range(NV):
                        ds = pl.ds(j * L, L)
                        m_old = macc[b, ds]
                        m_new = jnp.maximum(m_old, m_loc[j])
                        sacc[b, ds] = (sacc[b, ds] * jnp.exp(m_old - m_new)
                                       + s_loc[j] * jnp.exp(m_loc[j] - m_new))
                        macc[b, ds] = m_new

            return 0

        lax.fori_loop(0, NSEG, p1_seg, 0)

        nxt = c + NBUF

        @pl.when(nxt < NCHUNK)
        def _():
            in_copy(nxt, slot).start()

        return 0

    with jax.named_scope("phase1"):
        lax.fori_loop(0, NCHUNK, p1_chunk, 0)

    for k in range(3):
        in_copy(jnp.int32(k), jnp.int32(k)).start()

    sc_merge = jax.named_scope("scmerge")
    sc_merge.__enter__()
    pltpu.sync_copy(macc, shm.at[cgl, rg])
    pltpu.sync_copy(sacc, shs.at[cgl, rg])
    plsc.subcore_barrier()

    def init_g(b, _):
        for j in range(NV):
            gm[b, pl.ds(j * L, L)] = jnp.full((L,), _NEG, jnp.float32)
            gs[b, pl.ds(j * L, L)] = jnp.zeros((L,), jnp.float32)
        return 0

    lax.fori_loop(0, NSEG, init_g, 0)

    for w in range(NRG):
        pltpu.sync_copy(shm.at[cgl, w], tmp_m)
        pltpu.sync_copy(shs.at[cgl, w], tmp_s)

        def merge_b(b, _):
            for j in range(NV):
                ds = pl.ds(j * L, L)
                m_old = gm[b, ds]
                m_w = tmp_m[b, ds]
                m_new = jnp.maximum(m_old, m_w)
                gs[b, ds] = (gs[b, ds] * jnp.exp(m_old - m_new)
                             + tmp_s[b, ds] * jnp.exp(m_w - m_new))
                gm[b, ds] = m_new
            return 0

        lax.fori_loop(0, NSEG, merge_b, 0)

    def mk_lse(b, _):
        for j in range(NV):
            ds = pl.ds(j * L, L)
            lse[b, ds] = gm[b, ds] + _log16(gs[b, ds])
        return 0

    lax.fori_loop(0, NSEG, mk_lse, 0)
    sc_merge.__exit__(None, None, None)

    def p2_chunk(c, _):
        slot = lax.rem(c, jnp.int32(3))
        oslot = slot + 3
        in_copy(c, slot).wait()

        @pl.when(c >= 3)
        def _():
            out_copy(c - 3, oslot).wait()

        base = row0 + c * CH
        src = bufs.at[slot]
        dst = bufs.at[oslot]

        def p2_seg(b, _):
            lo, hi = _seg_bounds(ps_ref, b, base)

            @pl.when(lo < hi)
            def _():
                lv = tuple(lse[b, pl.ds(j * L, L)] for j in range(NV))

                def apply(i):
                    for j in range(NV):
                        ds = pl.ds(j * L, L)
                        dst[i, ds] = src[i, ds] - lv[j]

                plsc.parallel_loop(lo, hi, 1, unroll=2)(apply)

            return 0

        lax.fori_loop(0, NSEG, p2_seg, 0)
        out_copy(c, oslot).start()

        nxt = c + 3

        @pl.when(nxt < NCHUNK)
        def _():
            in_copy(nxt, slot).start()

        return 0

    with jax.named_scope("phase2"):
        lax.fori_loop(0, NCHUNK, p2_chunk, 0)

    for k in range(3):
        c = NCHUNK - 3 + k
        out_copy(jnp.int32(c), jnp.int32(c % 3 + 3)).wait()


_sc_call = functools.partial(
    pl.kernel,
    out_type=jax.ShapeDtypeStruct((TOTAL, D), jnp.float32),
    mesh=plsc.VectorSubcoreMesh(core_axis_name="c", subcore_axis_name="s"),
    scratch_types=[
        pltpu.VMEM((NBUF, CH, COLS), jnp.float32),
        pltpu.VMEM((32,), jnp.int32),
        pltpu.VMEM((NSEG, COLS), jnp.float32),
        pltpu.VMEM((NSEG, COLS), jnp.float32),
        pltpu.VMEM((NSEG, COLS), jnp.float32),
        pltpu.VMEM((NSEG, COLS), jnp.float32),
        pltpu.VMEM((NSEG, COLS), jnp.float32),
        pltpu.VMEM((NSEG, COLS), jnp.float32),
        pltpu.VMEM((NSEG, COLS), jnp.float32),
        pltpu.VMEM_SHARED((2, NRG, NSEG, COLS), jnp.float32),
        pltpu.VMEM_SHARED((2, NRG, NSEG, COLS), jnp.float32),
        pltpu.SemaphoreType.DMA((NBUF,)),
        pltpu.SemaphoreType.DMA((3,)),
    ],
    compiler_params=pltpu.CompilerParams(needs_layout_passes=False),
)(_body)


@jax.jit
def kernel(logits, prefix_sum):
    return _sc_call(logits, prefix_sum)


# speedup vs baseline: 1.2265x; 1.0822x over previous
import functools

import jax
import jax.numpy as jnp
from jax import lax
from jax.experimental import pallas as pl
from jax.experimental.pallas import tpu as pltpu
from jax.experimental.pallas import tpu_sc as plsc

TOTAL = 32768
D = 512
NSEG = 16
L = 16
NCORE = 2
NSUB = 16
COLS = 128
NV = COLS // L
NRG = 8
RROWS = TOTAL // NRG
CH = 128
NCHUNK = RROWS // CH
NBUF = 6

_NEG = -3e38
_LN2 = 0.6931471805599453


def _log16(s):
    xb = lax.bitcast_convert_type(s, jnp.int32)
    e = ((xb >> 23) & 0xFF) - 127
    mb = (xb & 0x7FFFFF) | 0x3F800000
    m = lax.bitcast_convert_type(mb, jnp.float32)
    big = m > jnp.float32(1.4142135)
    m = jnp.where(big, m * jnp.float32(0.5), m)
    e = e + jnp.where(big, jnp.int32(1), jnp.int32(0))
    t = (m - jnp.float32(1.0)) / (m + jnp.float32(1.0))
    t2 = t * t
    p = jnp.float32(1.0) + t2 * (
        jnp.float32(1.0 / 3.0)
        + t2 * (jnp.float32(1.0 / 5.0) + t2 * jnp.float32(1.0 / 7.0))
    )
    return e.astype(jnp.float32) * jnp.float32(_LN2) + jnp.float32(2.0) * t * p


def _seg_bounds(ps_ref, b, base):
    v = ps_ref[pl.ds(b, L)]
    lo = jnp.clip(v[0] - base, 0, CH)
    hi = jnp.clip(v[1] - base, 0, CH)
    return lo, hi


def _row_max(buf, lo, hi):
    neg = tuple(jnp.full((L,), _NEG, jnp.float32) for _ in range(NV))

    def body(i, c):
        return tuple(
            jnp.maximum(c[j], buf[i, pl.ds(j * L, L)]) for j in range(NV)
        )

    return plsc.parallel_loop(lo, hi, 1, unroll=2, carry=neg)(body)


def _row_sumexp(buf, lo, hi, ms):
    zero = tuple(jnp.zeros((L,), jnp.float32) for _ in range(NV))

    def body(i, c):
        return tuple(
            c[j] + jnp.exp(buf[i, pl.ds(j * L, L)] - ms[j]) for j in range(NV)
        )

    return plsc.parallel_loop(lo, hi, 1, unroll=2, carry=zero)(body)


def _body(logits_hbm, ps_hbm, out_hbm, bufs, ps_ref, macc, sacc, gm, gs, lse,
          tmp_m, tmp_s, shm, shs, in_sems, out_sems):
    core = lax.axis_index("c")
    sid = lax.axis_index("s")
    cgl = sid // NRG
    rg = lax.rem(sid, jnp.int32(NRG))
    col0 = (core * 2 + cgl) * COLS
    row0 = rg * RROWS

    pltpu.sync_copy(ps_hbm, ps_ref.at[pl.ds(0, NSEG + 1)])

    def in_copy(c, slot):
        return pltpu.make_async_copy(
            logits_hbm.at[pl.ds(row0 + c * CH, CH), pl.ds(col0, COLS)],
            bufs.at[slot],
            in_sems.at[slot],
        )

    def out_copy(c, slot):
        return pltpu.make_async_copy(
            bufs.at[slot],
            out_hbm.at[pl.ds(row0 + c * CH, CH), pl.ds(col0, COLS)],
            out_sems.at[slot - 3],
        )

    def init_b(b, _):
        for j in range(NV):
            macc[b, pl.ds(j * L, L)] = jnp.full((L,), _NEG, jnp.float32)
            sacc[b, pl.ds(j * L, L)] = jnp.zeros((L,), jnp.float32)
        return 0

    lax.fori_loop(0, NSEG, init_b, 0)

    for k in range(NBUF):
        in_copy(jnp.int32(k), jnp.int32(k)).start()

    def p1_chunk(c, _):
        slot = lax.rem(c, jnp.int32(NBUF))
        in_copy(c, slot).wait()
        base = row0 + c * CH
        buf = bufs.at[slot]

        def p1_seg(b, _):
            lo, hi = _seg_bounds(ps_ref, b, base)

            @pl.when(lo < hi)
            def _():
                msh = tuple(
                    jnp.where(macc[b, pl.ds(j * L, L)] < jnp.float32(-1e30),
                              buf[lo, pl.ds(j * L, L)],
                              macc[b, pl.ds(j * L, L)])
                    for j in range(NV)
                )
                s_fast = _row_sumexp(buf, lo, hi, msh)
                overv = s_fast[0]
                for j in range(1, NV):
                    overv = jnp.maximum(overv, s_fast[j])
                nbad = plsc.all_reduce_population_count(
                    overv > jnp.float32(1e35))
                bad = nbad[0] > 0

                @pl.when(jnp.logical_not(bad))
                def _():
                    for j in range(NV):
                        ds = pl.ds(j * L, L)
                        m_old = macc[b, ds]
                        m_new = jnp.maximum(m_old, msh[j])
                        sacc[b, ds] = (sacc[b, ds] * jnp.exp(m_old - m_new)
                                       + s_fast[j] * jnp.exp(msh[j] - m_new))
                        macc[b, ds] = m_new

                @pl.when(bad)
                def _():
                    m_loc = _row_max(buf, lo, hi)
                    s_loc = _row_sumexp(buf, lo, hi, m_loc)
                    for j in range(NV):
                        ds = pl.ds(j * L, L)
                        m_old = macc[b, ds]
                        m_new = jnp.maximum(m_old, m_loc[j])
                        sacc[b, ds] = (sacc[b, ds] * jnp.exp(m_old - m_new)
                                       + s_loc[j] * jnp.exp(m_loc[j] - m_new))
                        macc[b, ds] = m_new

            return 0

        lax.fori_loop(0, NSEG, p1_seg, 0)

        nxt = c + NBUF

        @pl.when(nxt < NCHUNK)
        def _():
            in_copy(nxt, slot).start()

        return 0

    with jax.named_scope("phase1"):
        lax.fori_loop(0, NCHUNK, p1_chunk, 0)

    for k in range(3):
        in_copy(jnp.int32(k), jnp.int32(k)).start()

    sc_merge = jax.named_scope("scmerge")
    sc_merge.__enter__()
    pltpu.sync_copy(macc, shm.at[cgl, rg])
    pltpu.sync_copy(sacc, shs.at[cgl, rg])
    plsc.subcore_barrier()

    def init_g(b, _):
        for j in range(NV):
            gm[b, pl.ds(j * L, L)] = jnp.full((L,), _NEG, jnp.float32)
            gs[b, pl.ds(j * L, L)] = jnp.zeros((L,), jnp.float32)
        return 0

    lax.fori_loop(0, NSEG, init_g, 0)

    for w in range(NRG):
        pltpu.sync_copy(shm.at[cgl, w], tmp_m)
        pltpu.sync_copy(shs.at[cgl, w], tmp_s)

        def merge_b(b, _):
            for j in range(NV):
                ds = pl.ds(j * L, L)
                m_old = gm[b, ds]
                m_w = tmp_m[b, ds]
                m_new = jnp.maximum(m_old, m_w)
                gs[b, ds] = (gs[b, ds] * jnp.exp(m_old - m_new)
                             + tmp_s[b, ds] * jnp.exp(m_w - m_new))
                gm[b, ds] = m_new
            return 0

        lax.fori_loop(0, NSEG, merge_b, 0)

    def mk_lse(b, _):
        for j in range(NV):
            ds = pl.ds(j * L, L)
            lse[b, ds] = gm[b, ds] + _log16(gs[b, ds])
        return 0

    lax.fori_loop(0, NSEG, mk_lse, 0)
    sc_merge.__exit__(None, None, None)

    def p2_chunk(c, _):
        slot = lax.rem(c, jnp.int32(3))
        oslot = slot + 3
        in_copy(c, slot).wait()

        @pl.when(c >= 3)
        def _():
            out_copy(c - 3, oslot).wait()

        base = row0 + c * CH
        src = bufs.at[slot]
        dst = bufs.at[oslot]

        def p2_seg(b, _):
            lo, hi = _seg_bounds(ps_ref, b, base)

            @pl.when(lo < hi)
            def _():
                lv = tuple(lse[b, pl.ds(j * L, L)] for j in range(NV))

                def apply(i):
                    for j in range(NV):
                        ds = pl.ds(j * L, L)
                        dst[i, ds] = src[i, ds] - lv[j]

                plsc.parallel_loop(lo, hi, 1, unroll=2)(apply)

            return 0

        lax.fori_loop(0, NSEG, p2_seg, 0)
        out_copy(c, oslot).start()

        nxt = c + 3

        @pl.when(nxt < NCHUNK)
        def _():
            in_copy(nxt, slot).start()

        return 0

    with jax.named_scope("phase2"):
        lax.fori_loop(0, NCHUNK, p2_chunk, 0)

    for k in range(3):
        c = NCHUNK - 3 + k
        out_copy(jnp.int32(c), jnp.int32(c % 3 + 3)).wait()


_sc_call = functools.partial(
    pl.kernel,
    out_type=jax.ShapeDtypeStruct((TOTAL, D), jnp.float32),
    mesh=plsc.VectorSubcoreMesh(core_axis_name="c", subcore_axis_name="s"),
    scratch_types=[
        pltpu.VMEM((NBUF, CH, COLS), jnp.float32),
        pltpu.VMEM((32,), jnp.int32),
        pltpu.VMEM((NSEG, COLS), jnp.float32),
        pltpu.VMEM((NSEG, COLS), jnp.float32),
        pltpu.VMEM((NSEG, COLS), jnp.float32),
        pltpu.VMEM((NSEG, COLS), jnp.float32),
        pltpu.VMEM((NSEG, COLS), jnp.float32),
        pltpu.VMEM((NSEG, COLS), jnp.float32),
        pltpu.VMEM((NSEG, COLS), jnp.float32),
        pltpu.VMEM_SHARED((2, NRG, NSEG, COLS), jnp.float32),
        pltpu.VMEM_SHARED((2, NRG, NSEG, COLS), jnp.float32),
        pltpu.SemaphoreType.DMA((NBUF,)),
        pltpu.SemaphoreType.DMA((3,)),
    ],
    compiler_params=pltpu.CompilerParams(needs_layout_passes=False),
)(_body)


@jax.jit
def kernel(logits, prefix_sum):
    return _sc_call(logits, prefix_sum)
